# baseline (speedup 1.0000x reference)
import functools

import jax
import jax.numpy as jnp
from jax import lax
from jax.experimental import pallas as pl
from jax.experimental.pallas import tpu as pltpu
from jax.experimental.pallas import tpu_sc as plsc

EMB = 512
P = 1024
B = 8
LANES = 16


def _idx_body(z_ref, w_ref, idx_ref):
    z = z_ref[0]
    w = w_ref[...]
    cross = lax.dot_general(z, w, (((0,), (0,)), ((), ())),
                            preferred_element_type=jnp.float32)
    z2 = jnp.sum(z * z, axis=0)
    w2 = jnp.sum(w * w, axis=0)
    dist = (z2[:, None] + w2[None, :]) - 2.0 * cross
    m = jnp.min(dist, axis=1, keepdims=True)
    kio = lax.broadcasted_iota(jnp.int32, (P, EMB), 1)
    idx_ref[0, 0] = jnp.min(jnp.where(dist == m, kio, EMB), axis=1)


def _nearest_idx(z3, emb_weight):
    return pl.pallas_call(
        _idx_body,
        grid=(B,),
        in_specs=[
            pl.BlockSpec((1, EMB, P), lambda b: (b, 0, 0)),
            pl.BlockSpec((EMB, EMB), lambda b: (0, 0)),
        ],
        out_specs=pl.BlockSpec((1, 1, P), lambda b: (b, 0, 0)),
        out_shape=jax.ShapeDtypeStruct((B, 1, P), jnp.int32),
    )(z3, emb_weight)


_MESH = plsc.VectorSubcoreMesh(core_axis_name="c", subcore_axis_name="s",
                               num_cores=2, num_subcores=16)
_NW = _MESH.num_cores * _MESH.num_subcores
_DPW = EMB // _NW


@functools.partial(
    pl.kernel,
    out_type=jax.ShapeDtypeStruct((B, EMB, P), jnp.float32),
    mesh=_MESH,
    scratch_types=[
        pltpu.VMEM((_DPW, EMB), jnp.float32),
        pltpu.VMEM((P,), jnp.int32),
        pltpu.VMEM((_DPW, P), jnp.float32),
    ],
    compiler_params=pltpu.CompilerParams(use_tc_tiling_on_sc=False,
                                         needs_layout_passes=False),
)
def _sc_gather(w_hbm, idx_hbm, out_hbm, w_v, idx_v, out_v):
    wid = lax.axis_index("s") * _MESH.num_cores + lax.axis_index("c")
    d0 = wid * _DPW
    pltpu.sync_copy(w_hbm.at[pl.ds(d0, _DPW)], w_v)
    for b in range(B):
        pltpu.sync_copy(idx_hbm.at[b, 0], idx_v)

        def chunk(c, carry):
            ivec = idx_v[pl.ds(c * LANES, LANES)]
            for dl in range(_DPW):
                row = jnp.full((LANES,), dl, jnp.int32)
                out_v[dl, pl.ds(c * LANES, LANES)] = plsc.load_gather(
                    w_v, [row, ivec])
            return carry

        lax.fori_loop(0, P // LANES, chunk, 0)
        pltpu.sync_copy(out_v, out_hbm.at[b, pl.ds(d0, _DPW)])


def kernel(x, emb_weight):
    z3 = x.reshape(B, EMB, P)
    idx3 = _nearest_idx(z3, emb_weight)
    q3 = _sc_gather(emb_weight, idx3)
    return q3, x, q3.reshape(x.shape)


# speedup vs baseline: 1.0996x; 1.0996x over previous
import functools

import jax
import jax.numpy as jnp
from jax import lax
from jax.experimental import pallas as pl
from jax.experimental.pallas import tpu as pltpu
from jax.experimental.pallas import tpu_sc as plsc

EMB = 512
P = 1024
B = 8
LANES = 16


def _idx_body(z_ref, w_ref, idx_ref, zc_ref):
    z = z_ref[0]
    w = w_ref[...]
    cross = lax.dot_general(z, w, (((0,), (0,)), ((), ())),
                            preferred_element_type=jnp.float32)
    z2 = jnp.sum(z * z, axis=0)
    w2 = jnp.sum(w * w, axis=0)
    dist = (z2[:, None] + w2[None, :]) - 2.0 * cross
    m = jnp.min(dist, axis=1, keepdims=True)
    kio = lax.broadcasted_iota(jnp.int32, (P, EMB), 1)
    idx_ref[0, 0] = jnp.min(jnp.where(dist == m, kio, EMB), axis=1)
    zc_ref[0] = z


def _nearest_idx(z3, emb_weight):
    return pl.pallas_call(
        _idx_body,
        grid=(B,),
        in_specs=[
            pl.BlockSpec((1, EMB, P), lambda b: (b, 0, 0)),
            pl.BlockSpec((EMB, EMB), lambda b: (0, 0)),
        ],
        out_specs=[
            pl.BlockSpec((1, 1, P), lambda b: (b, 0, 0)),
            pl.BlockSpec((1, EMB, P), lambda b: (b, 0, 0)),
        ],
        out_shape=[
            jax.ShapeDtypeStruct((B, 1, P), jnp.int32),
            jax.ShapeDtypeStruct((B, EMB, P), jnp.float32),
        ],
    )(z3, emb_weight)


def _onehot_body(w_ref, idx_ref, out_ref):
    idx = idx_ref[0, 0]
    kio = lax.broadcasted_iota(jnp.int32, (P, EMB), 1)
    onehot = (kio == idx[:, None]).astype(jnp.float32)
    out_ref[0] = lax.dot_general(w_ref[...], onehot, (((1,), (1,)), ((), ())),
                                 precision=lax.Precision.HIGHEST,
                                 preferred_element_type=jnp.float32)


def _onehot_gather(emb_weight, idx3):
    return pl.pallas_call(
        _onehot_body,
        grid=(B,),
        in_specs=[
            pl.BlockSpec((EMB, EMB), lambda b: (0, 0)),
            pl.BlockSpec((1, 1, P), lambda b: (b, 0, 0)),
        ],
        out_specs=pl.BlockSpec((1, EMB, P), lambda b: (b, 0, 0)),
        out_shape=jax.ShapeDtypeStruct((B, EMB, P), jnp.float32),
    )(emb_weight, idx3)


_MESH = plsc.VectorSubcoreMesh(core_axis_name="c", subcore_axis_name="s",
                               num_cores=2, num_subcores=16)
_NW = _MESH.num_cores * _MESH.num_subcores
_DPW = EMB // _NW


@functools.partial(
    pl.kernel,
    out_type=jax.ShapeDtypeStruct((B, EMB, P), jnp.float32),
    mesh=_MESH,
    scratch_types=[
        pltpu.VMEM((_DPW, EMB), jnp.float32),
        pltpu.VMEM((B, 1, P), jnp.int32),
        pltpu.VMEM((2, _DPW, P), jnp.float32),
        pltpu.SemaphoreType.DMA,
        pltpu.SemaphoreType.DMA,
    ],
    compiler_params=pltpu.CompilerParams(use_tc_tiling_on_sc=False,
                                         needs_layout_passes=False),
)
def _sc_gather(w_hbm, idx_hbm, out_hbm, w_v, idx_v, out_v, sem0, sem1):
    wid = lax.axis_index("s") * _MESH.num_cores + lax.axis_index("c")
    d0 = wid * _DPW
    pltpu.sync_copy(w_hbm.at[pl.ds(d0, _DPW)], w_v)
    pltpu.sync_copy(idx_hbm, idx_v)
    sems = (sem0, sem1)
    pending = [None, None]
    for b in range(B):
        par = b % 2
        if pending[par] is not None:
            pending[par].wait()

        def chunk(c, carry, b=b, par=par):
            ivec = idx_v[b, 0, pl.ds(c * LANES, LANES)]
            for dl in range(_DPW):
                row = jnp.full((LANES,), dl, jnp.int32)
                out_v[par, dl, pl.ds(c * LANES, LANES)] = plsc.load_gather(
                    w_v, [row, ivec])
            return carry

        lax.fori_loop(0, P // LANES, chunk, 0)
        pending[par] = pltpu.make_async_copy(
            out_v.at[par], out_hbm.at[b, pl.ds(d0, _DPW)], sems[par])
        pending[par].start()
    for par in range(2):
        if pending[par] is not None:
            pending[par].wait()


def kernel(x, emb_weight):
    z3 = x.reshape(B, EMB, P)
    idx3, zc = _nearest_idx(z3, emb_weight)
    q3 = _sc_gather(emb_weight, idx3)
    q4 = _onehot_gather(emb_weight, idx3)
    return q3, zc.reshape(x.shape), q4.reshape(x.shape)
